# Initial kernel scaffold; baseline (speedup 1.0000x reference)
import jax
import jax.numpy as jnp
from jax.experimental import pallas as pl


def kernel(edge_index, adj_values, user_emb, item_emb):
    raise NotImplementedError("write your pallas kernel here")


# speedup vs baseline: 1.6481x; 1.6481x over previous
import jax
import jax.numpy as jnp
from jax import lax
from jax.experimental import pallas as pl
from jax.experimental.pallas import tpu as pltpu
from jax.experimental.pallas import tpu_sc as plsc

L = 16
SRC_BITS = 14
SCAN_CH = 3200
FLUSH = 2048
STAGE = FLUSH + 256
CHUNK = 128


def _route_kernel(nc, ns, rows_per, ep, cap):
  nw = nc * ns
  mesh = plsc.VectorSubcoreMesh(core_axis_name="c", subcore_axis_name="s")

  def body(dst_h, src_h, w_h, pk_o, wr_o, cnt_o,
           dstv, srcv, wv, stg_pk, stg_w, cntv):
    wid = lax.axis_index("s") * nc + lax.axis_index("c")
    lo = wid * rows_per
    lanes = lax.iota(jnp.int32, L)

    def append(cnt, m, pk, w):
      mi = m.astype(jnp.int32)
      cs = plsc.cumsum(mi)
      pos = jnp.maximum(cnt + cs - 1, 0)
      plsc.store_scatter(stg_pk, [pos], pk, mask=m)
      plsc.store_scatter(stg_w, [pos], w, mask=m)
      return cnt + jnp.max(cs)

    def maybe_flush(cnt, total):
      do = cnt >= FLUSH

      @pl.when(do)
      def _():
        t = pl.multiple_of(total, FLUSH)
        pltpu.sync_copy(stg_pk.at[pl.ds(0, FLUSH)],
                        pk_o.at[wid, pl.ds(t, FLUSH)])
        pltpu.sync_copy(stg_w.at[pl.ds(0, FLUSH)],
                        wr_o.at[wid, pl.ds(t, FLUSH)])
        stg_pk[pl.ds(0, L)] = stg_pk[pl.ds(FLUSH, L)]
        stg_w[pl.ds(0, L)] = stg_w[pl.ds(FLUSH, L)]

      cnt = jnp.where(do, cnt - FLUSH, cnt)
      total = jnp.where(do, total + FLUSH, total)
      return cnt, total

    def group(gi, carry):
      cnt, total = carry
      off = pl.multiple_of(gi * L, L)
      d = dstv[pl.ds(off, L)]
      s = srcv[pl.ds(off, L)]
      w = wv[pl.ds(off, L)]
      m = (d >= lo) & (d < lo + rows_per)
      pk = s | ((d - lo) << SRC_BITS)
      cnt = append(cnt, m, pk, w)
      return maybe_flush(cnt, total)

    def chunk(ci, carry):
      base = pl.multiple_of(ci * SCAN_CH, SCAN_CH)
      pltpu.sync_copy(dst_h.at[pl.ds(base, SCAN_CH)], dstv)
      pltpu.sync_copy(src_h.at[pl.ds(base, SCAN_CH)], srcv)
      pltpu.sync_copy(w_h.at[pl.ds(base, SCAN_CH)], wv)
      return lax.fori_loop(0, SCAN_CH // L, group, carry)

    cnt, total = lax.fori_loop(0, ep // SCAN_CH, chunk,
                               (jnp.int32(0), jnp.int32(0)))

    r = lax.rem(total + cnt, jnp.int32(CHUNK))
    npad = lax.rem(jnp.int32(CHUNK) - r, jnp.int32(CHUNK))
    zpk = jnp.zeros((L,), jnp.int32)
    zw = jnp.zeros((L,), jnp.float32)

    def pad_group(gi, carry):
      cnt, rem = carry
      take = jnp.minimum(rem, L)
      m = lanes < take
      cnt = append(cnt, m, zpk, zw)
      return cnt, rem - take

    cnt, _ = lax.fori_loop(0, CHUNK // L, pad_group, (cnt, npad))

    def fflush(fi, _):
      o = pl.multiple_of(fi * CHUNK, CHUNK)
      t = pl.multiple_of(total + o, CHUNK)
      pltpu.sync_copy(stg_pk.at[pl.ds(o, CHUNK)],
                      pk_o.at[wid, pl.ds(t, CHUNK)])
      pltpu.sync_copy(stg_w.at[pl.ds(o, CHUNK)],
                      wr_o.at[wid, pl.ds(t, CHUNK)])
      return 0

    lax.fori_loop(0, cnt // CHUNK, fflush, 0)
    cntv[...] = jnp.full((L,), total + cnt, jnp.int32)
    pltpu.sync_copy(cntv, cnt_o.at[wid])

  return pl.kernel(
      body,
      out_type=[jax.ShapeDtypeStruct((nw, cap), jnp.int32),
                jax.ShapeDtypeStruct((nw, cap), jnp.float32),
                jax.ShapeDtypeStruct((nw, L), jnp.int32)],
      mesh=mesh,
      scratch_types=[pltpu.VMEM((SCAN_CH,), jnp.int32),
                     pltpu.VMEM((SCAN_CH,), jnp.int32),
                     pltpu.VMEM((SCAN_CH,), jnp.float32),
                     pltpu.VMEM((STAGE,), jnp.int32),
                     pltpu.VMEM((STAGE,), jnp.float32),
                     pltpu.VMEM((L,), jnp.int32)],
      compiler_params=pltpu.CompilerParams(needs_layout_passes=False),
  )


def _layer_kernel(nc, ns, rows_per, n_pad, cap, dim):
  dg = dim // L
  mesh = plsc.VectorSubcoreMesh(core_axis_name="c", subcore_axis_name="s")

  def body(x_h, pk_h, wr_h, cnt_h, xo_h, pkv, wv, idxv, rows, acc, cntv, sem):
    wid = lax.axis_index("s") * nc + lax.axis_index("c")
    pltpu.sync_copy(cnt_h.at[wid], cntv)
    kp = jnp.max(cntv[...])

    zero = jnp.zeros((L,), jnp.float32)

    def zrow(rr, _):
      for u in range(dg):
        acc[rr, pl.ds(u * L, L)] = zero
      return 0

    lax.fori_loop(0, rows_per, zrow, 0)

    def chunk(ci, _):
      base = pl.multiple_of(ci * CHUNK, CHUNK)
      pltpu.sync_copy(pk_h.at[wid, pl.ds(base, CHUNK)], pkv)
      pltpu.sync_copy(wr_h.at[wid, pl.ds(base, CHUNK)], wv)
      for g in range(CHUNK // L):
        sl = pl.ds(g * L, L)
        idxv[sl] = pkv[sl] & ((1 << SRC_BITS) - 1)
      pltpu.async_copy(x_h.at[idxv], rows, sem).wait()

      def grp(gi, _):
        goff = pl.multiple_of(gi * L, L)
        pk16 = pkv[pl.ds(goff, L)]
        w16 = wv[pl.ds(goff, L)]
        for j in range(L):
          pkj = pk16[j]
          wj = w16[j]
          dl = pkj >> SRC_BITS
          for u in range(dg):
            sl = pl.ds(u * L, L)
            plsc.addupdate(acc.at[dl, sl], rows[gi * L + j, sl] * wj)
        return 0

      lax.fori_loop(0, CHUNK // L, grp, 0)
      return 0

    lax.fori_loop(0, kp // CHUNK, chunk, 0)
    row0 = pl.multiple_of(wid * rows_per, 8)
    pltpu.sync_copy(acc, xo_h.at[pl.ds(row0, rows_per)])

  return pl.kernel(
      body,
      out_type=jax.ShapeDtypeStruct((n_pad, dim), jnp.float32),
      mesh=mesh,
      scratch_types=[pltpu.VMEM((CHUNK,), jnp.int32),
                     pltpu.VMEM((CHUNK,), jnp.float32),
                     pltpu.VMEM((CHUNK,), jnp.int32),
                     pltpu.VMEM((CHUNK, dim), jnp.float32),
                     pltpu.VMEM((rows_per, dim), jnp.float32),
                     pltpu.VMEM((L,), jnp.int32),
                     pltpu.SemaphoreType.DMA],
      compiler_params=pltpu.CompilerParams(needs_layout_passes=False),
  )


def _mean4(base, a, b, c, blk):
  n, dim = base.shape

  def body(b0, b1, b2, b3, o):
    o[...] = (b0[...] + b1[...] + b2[...] + b3[...]) * 0.25

  spec = pl.BlockSpec((blk, dim), lambda i: (i, 0))
  return pl.pallas_call(
      body,
      grid=(n // blk,),
      in_specs=[spec] * 4,
      out_specs=spec,
      out_shape=jax.ShapeDtypeStruct((n, dim), jnp.float32),
  )(base, a, b, c)


def kernel(edge_index, adj_values, user_emb, item_emb):
  nu, dim = user_emb.shape
  ni = item_emb.shape[0]
  n = nu + ni
  e = edge_index.shape[1]

  info = plsc.get_sparse_core_info()
  nc, ns = info.num_cores, info.num_subcores
  nw = nc * ns
  rows_per = (-(-n // nw) + 7) // 8 * 8
  n_pad = nw * rows_per
  ep = -(-e // SCAN_CH) * SCAN_CH
  cap = -(-ep // CHUNK) * CHUNK + CHUNK

  dst = edge_index[0]
  src = edge_index[1]
  w = adj_values
  if ep != e:
    pad = ep - e
    dst = jnp.concatenate([dst, jnp.zeros((pad,), dst.dtype)])
    src = jnp.concatenate([src, jnp.zeros((pad,), src.dtype)])
    w = jnp.concatenate([w, jnp.zeros((pad,), w.dtype)])

  x0 = jnp.concatenate([user_emb, item_emb], axis=0)
  x0p = jnp.pad(x0, ((0, n_pad - n), (0, 0)))

  route = _route_kernel(nc, ns, rows_per, ep, cap)
  pk, wr, cnts = route(dst, src, w)

  layer = _layer_kernel(nc, ns, rows_per, n_pad, cap, dim)
  x1 = layer(x0p, pk, wr, cnts)
  x2 = layer(x1, pk, wr, cnts)
  x3 = layer(x2, pk, wr, cnts)

  blk_u = 1000 if nu % 1000 == 0 else 8
  blk_i = 1000 if ni % 1000 == 0 else 8
  x1u = lax.slice(x1, (0, 0), (nu, dim))
  x2u = lax.slice(x2, (0, 0), (nu, dim))
  x3u = lax.slice(x3, (0, 0), (nu, dim))
  x1i = lax.slice(x1, (nu, 0), (n, dim))
  x2i = lax.slice(x2, (nu, 0), (n, dim))
  x3i = lax.slice(x3, (nu, 0), (n, dim))
  users = _mean4(user_emb, x1u, x2u, x3u, blk_u)
  items = _mean4(item_emb, x1i, x2i, x3i, blk_i)
  return users, items


# speedup vs baseline: 4.4261x; 2.6855x over previous
import jax
import jax.numpy as jnp
from jax import lax
from jax.experimental import pallas as pl
from jax.experimental.pallas import tpu as pltpu
from jax.experimental.pallas import tpu_sc as plsc

L = 16
SRC_BITS = 14
SCAN_CH = 3200
FLUSH = 2048
STAGE = FLUSH + 256
CHUNK = 128


def _route_kernel(nc, ns, rows_per, ep, cap):
  nw = nc * ns
  mesh = plsc.VectorSubcoreMesh(core_axis_name="c", subcore_axis_name="s")

  def body(dst_h, src_h, w_h, pk_o, wr_o, cnt_o,
           dstv, srcv, wv, stg_pk, stg_w, cntv, sem0, sem1):
    wid = lax.axis_index("s") * nc + lax.axis_index("c")
    lo = wid * rows_per
    lanes = lax.iota(jnp.int32, L)

    def append(cnt, m, pk, w):
      inc = plsc.all_reduce_population_count(m)[0]
      plsc.store_compressed(stg_pk.at[pl.ds(cnt, L)], pk, mask=m)
      plsc.store_compressed(stg_w.at[pl.ds(cnt, L)], w, mask=m)
      return cnt + inc

    def maybe_flush(cnt, total):
      do = cnt >= FLUSH

      @pl.when(do)
      def _():
        t = pl.multiple_of(total, FLUSH)
        pltpu.sync_copy(stg_pk.at[pl.ds(0, FLUSH)],
                        pk_o.at[wid, pl.ds(t, FLUSH)])
        pltpu.sync_copy(stg_w.at[pl.ds(0, FLUSH)],
                        wr_o.at[wid, pl.ds(t, FLUSH)])
        stg_pk[pl.ds(0, L)] = stg_pk[pl.ds(FLUSH, L)]
        stg_w[pl.ds(0, L)] = stg_w[pl.ds(FLUSH, L)]

      cnt = jnp.where(do, cnt - FLUSH, cnt)
      total = jnp.where(do, total + FLUSH, total)
      return cnt, total

    def group_nf(bo, gi, cnt):
      off = pl.multiple_of(bo + gi * L, L)
      d = dstv[pl.ds(off, L)]
      s = srcv[pl.ds(off, L)]
      w = wv[pl.ds(off, L)]
      m = (d >= lo) & (d < lo + rows_per)
      pk = s | ((d - lo) << SRC_BITS)
      return append(cnt, m, pk, w)

    nchunks = ep // SCAN_CH
    sems = (sem0, sem1)

    def start_load(ci, b):
      base = pl.multiple_of(ci * SCAN_CH, SCAN_CH)
      bo = b * SCAN_CH
      pltpu.async_copy(dst_h.at[pl.ds(base, SCAN_CH)],
                       dstv.at[pl.ds(bo, SCAN_CH)], sems[b])
      pltpu.async_copy(src_h.at[pl.ds(base, SCAN_CH)],
                       srcv.at[pl.ds(bo, SCAN_CH)], sems[b])
      pltpu.async_copy(w_h.at[pl.ds(base, SCAN_CH)],
                       wv.at[pl.ds(bo, SCAN_CH)], sems[b])

    def wait_load(ci, b):
      base = pl.multiple_of(ci * SCAN_CH, SCAN_CH)
      bo = b * SCAN_CH
      pltpu.make_async_copy(dst_h.at[pl.ds(base, SCAN_CH)],
                            dstv.at[pl.ds(bo, SCAN_CH)], sems[b]).wait()
      pltpu.make_async_copy(src_h.at[pl.ds(base, SCAN_CH)],
                            srcv.at[pl.ds(bo, SCAN_CH)], sems[b]).wait()
      pltpu.make_async_copy(w_h.at[pl.ds(base, SCAN_CH)],
                            wv.at[pl.ds(bo, SCAN_CH)], sems[b]).wait()

    def scan_chunk(b, carry):
      bo = b * SCAN_CH

      def block(bi, carry):
        cnt, total = carry
        for k in range(8):
          cnt = group_nf(bo, bi * 8 + k, cnt)
        return maybe_flush(cnt, total)

      return lax.fori_loop(0, SCAN_CH // L // 8, block, carry)

    start_load(jnp.int32(0), 0)

    def pair(p, carry):
      c0 = 2 * p
      c1 = c0 + 1

      @pl.when(c1 < nchunks)
      def _():
        start_load(c1, 1)

      wait_load(c0, 0)
      carry = scan_chunk(0, carry)

      def odd(carry):
        @pl.when(c1 + 1 < nchunks)
        def _():
          start_load(c1 + 1, 0)

        wait_load(c1, 1)
        return scan_chunk(1, carry)

      carry = lax.cond(c1 < nchunks, odd, lambda c: c, carry)
      return carry

    cnt, total = lax.fori_loop(0, (nchunks + 1) // 2, pair,
                               (jnp.int32(0), jnp.int32(0)))

    r = lax.rem(total + cnt, jnp.int32(CHUNK))
    npad = lax.rem(jnp.int32(CHUNK) - r, jnp.int32(CHUNK))
    zpk = jnp.zeros((L,), jnp.int32)
    zw = jnp.zeros((L,), jnp.float32)

    def pad_group(gi, carry):
      cnt, rem = carry
      take = jnp.minimum(rem, L)
      m = lanes < take
      cnt = append(cnt, m, zpk, zw)
      return cnt, rem - take

    cnt, _ = lax.fori_loop(0, CHUNK // L, pad_group, (cnt, npad))

    def fflush(fi, _):
      o = pl.multiple_of(fi * CHUNK, CHUNK)
      t = pl.multiple_of(total + o, CHUNK)
      pltpu.sync_copy(stg_pk.at[pl.ds(o, CHUNK)],
                      pk_o.at[wid, pl.ds(t, CHUNK)])
      pltpu.sync_copy(stg_w.at[pl.ds(o, CHUNK)],
                      wr_o.at[wid, pl.ds(t, CHUNK)])
      return 0

    lax.fori_loop(0, cnt // CHUNK, fflush, 0)
    cntv[...] = jnp.full((L,), total + cnt, jnp.int32)
    pltpu.sync_copy(cntv, cnt_o.at[wid])

  return pl.kernel(
      body,
      out_type=[jax.ShapeDtypeStruct((nw, cap), jnp.int32),
                jax.ShapeDtypeStruct((nw, cap), jnp.float32),
                jax.ShapeDtypeStruct((nw, L), jnp.int32)],
      mesh=mesh,
      scratch_types=[pltpu.VMEM((2 * SCAN_CH,), jnp.int32),
                     pltpu.VMEM((2 * SCAN_CH,), jnp.int32),
                     pltpu.VMEM((2 * SCAN_CH,), jnp.float32),
                     pltpu.VMEM((STAGE,), jnp.int32),
                     pltpu.VMEM((STAGE,), jnp.float32),
                     pltpu.VMEM((L,), jnp.int32),
                     pltpu.SemaphoreType.DMA,
                     pltpu.SemaphoreType.DMA],
      compiler_params=pltpu.CompilerParams(needs_layout_passes=False),
  )


def _layer_kernel(nc, ns, rows_per, n_pad, cap, dim):
  dg = dim // L
  mesh = plsc.VectorSubcoreMesh(core_axis_name="c", subcore_axis_name="s")

  def body(x_h, pk_h, wr_h, cnt_h, xo_h, pkv, wv, idxv, rows, acc, cntv,
           sem0, sem1):
    wid = lax.axis_index("s") * nc + lax.axis_index("c")
    pltpu.sync_copy(cnt_h.at[wid], cntv)
    kp = jnp.max(cntv[...])

    zero = jnp.zeros((L,), jnp.float32)

    @plsc.parallel_loop(0, rows_per)
    def _(rr):
      for u in range(dg):
        acc[rr, pl.ds(u * L, L)] = zero

    nch = kp // CHUNK
    sems = (sem0, sem1)

    def stage(c, b):
      base = pl.multiple_of(c * CHUNK, CHUNK)
      bo = b * CHUNK
      pltpu.sync_copy(pk_h.at[wid, pl.ds(base, CHUNK)],
                      pkv.at[pl.ds(bo, CHUNK)])
      pltpu.sync_copy(wr_h.at[wid, pl.ds(base, CHUNK)],
                      wv.at[pl.ds(bo, CHUNK)])
      for g in range(CHUNK // L):
        sl = pl.ds(bo + g * L, L)
        idxv[sl] = pkv[sl] & ((1 << SRC_BITS) - 1)
      pltpu.async_copy(x_h.at[idxv.at[pl.ds(bo, CHUNK)]],
                       rows.at[pl.ds(bo, CHUNK)], sems[b])

    def wait_gather(b):
      bo = b * CHUNK
      pltpu.make_async_copy(x_h.at[idxv.at[pl.ds(bo, CHUNK)]],
                            rows.at[pl.ds(bo, CHUNK)], sems[b]).wait()

    lanes = lax.iota(jnp.int32, L)
    cols = [lanes + u * L for u in range(dg)]

    def accum(b):
      bo = b * CHUNK

      @plsc.parallel_loop(0, CHUNK // L, unroll=2)
      def _(gi):
        goff = pl.multiple_of(gi * L, L)
        pk16 = pkv[pl.ds(bo + goff, L)]
        w16 = wv[pl.ds(bo + goff, L)]
        dl16 = pk16 >> SRC_BITS
        for j in range(L):
          jj = jnp.full((L,), j, jnp.int32)
          dlv = jnp.take_along_axis(dl16, jj, axis=0)
          wjv = jnp.take_along_axis(w16, jj, axis=0)
          r = bo + gi * L + j
          vals = [rows[r, pl.ds(u * L, L)] for u in range(dg)]
          for u in range(dg):
            plsc.addupdate_scatter(acc, [dlv, cols[u]], vals[u] * wjv)

    @pl.when(nch > 0)
    def _():
      stage(jnp.int32(0), 0)

      def pair(p, _):
        c0 = 2 * p
        c1 = c0 + 1

        @pl.when(c1 < nch)
        def _():
          stage(c1, 1)

        wait_gather(0)
        accum(0)

        @pl.when(c1 < nch)
        def _():
          @pl.when(c1 + 1 < nch)
          def _():
            stage(c1 + 1, 0)

          wait_gather(1)
          accum(1)

        return 0

      lax.fori_loop(0, (nch + 1) // 2, pair, 0)

    row0 = pl.multiple_of(wid * rows_per, 8)
    pltpu.sync_copy(acc, xo_h.at[pl.ds(row0, rows_per)])

  return pl.kernel(
      body,
      out_type=jax.ShapeDtypeStruct((n_pad, dim), jnp.float32),
      mesh=mesh,
      scratch_types=[pltpu.VMEM((2 * CHUNK,), jnp.int32),
                     pltpu.VMEM((2 * CHUNK,), jnp.float32),
                     pltpu.VMEM((2 * CHUNK,), jnp.int32),
                     pltpu.VMEM((2 * CHUNK, dim), jnp.float32),
                     pltpu.VMEM((rows_per, dim), jnp.float32),
                     pltpu.VMEM((L,), jnp.int32),
                     pltpu.SemaphoreType.DMA,
                     pltpu.SemaphoreType.DMA],
      compiler_params=pltpu.CompilerParams(needs_layout_passes=False),
  )


def _mean4(base, a, b, c, blk):
  n, dim = base.shape

  def body(b0, b1, b2, b3, o):
    o[...] = (b0[...] + b1[...] + b2[...] + b3[...]) * 0.25

  spec = pl.BlockSpec((blk, dim), lambda i: (i, 0))
  return pl.pallas_call(
      body,
      grid=(n // blk,),
      in_specs=[spec] * 4,
      out_specs=spec,
      out_shape=jax.ShapeDtypeStruct((n, dim), jnp.float32),
  )(base, a, b, c)


def kernel(edge_index, adj_values, user_emb, item_emb):
  nu, dim = user_emb.shape
  ni = item_emb.shape[0]
  n = nu + ni
  e = edge_index.shape[1]

  info = plsc.get_sparse_core_info()
  nc, ns = info.num_cores, info.num_subcores
  nw = nc * ns
  rows_per = (-(-n // nw) + 7) // 8 * 8
  n_pad = nw * rows_per
  ep = -(-e // SCAN_CH) * SCAN_CH
  cap = -(-ep // CHUNK) * CHUNK + CHUNK

  dst = edge_index[0]
  src = edge_index[1]
  w = adj_values
  if ep != e:
    pad = ep - e
    dst = jnp.concatenate([dst, jnp.zeros((pad,), dst.dtype)])
    src = jnp.concatenate([src, jnp.zeros((pad,), src.dtype)])
    w = jnp.concatenate([w, jnp.zeros((pad,), w.dtype)])

  x0 = jnp.concatenate([user_emb, item_emb], axis=0)
  x0p = jnp.pad(x0, ((0, n_pad - n), (0, 0)))

  route = _route_kernel(nc, ns, rows_per, ep, cap)
  pk, wr, cnts = route(dst, src, w)

  layer = _layer_kernel(nc, ns, rows_per, n_pad, cap, dim)
  x1 = layer(x0p, pk, wr, cnts)
  x2 = layer(x1, pk, wr, cnts)
  x3 = layer(x2, pk, wr, cnts)

  blk_u = 1000 if nu % 1000 == 0 else 8
  blk_i = 1000 if ni % 1000 == 0 else 8
  x1u = lax.slice(x1, (0, 0), (nu, dim))
  x2u = lax.slice(x2, (0, 0), (nu, dim))
  x3u = lax.slice(x3, (0, 0), (nu, dim))
  x1i = lax.slice(x1, (nu, 0), (n, dim))
  x2i = lax.slice(x2, (nu, 0), (n, dim))
  x3i = lax.slice(x3, (nu, 0), (n, dim))
  users = _mean4(user_emb, x1u, x2u, x3u, blk_u)
  items = _mean4(item_emb, x1i, x2i, x3i, blk_i)
  return users, items
